# baseline (speedup 1.0000x reference)
import jax
import jax.numpy as jnp
from jax import lax
from jax.experimental import pallas as pl
from jax.experimental.pallas import tpu as pltpu
from jax.experimental.pallas import tpu_sc as plsc

_N_CLUSTERS = 50
_CLUSTER = 2000
_TOT = 3200
_T = 128
_D = 8
_K = 64
_CPB = 5
_NBLK = _N_CLUSTERS // _CPB
_NW = 32
_RPW = _TOT // _NW


def _sc_gather(x_hbm, idx_hbm, out_hbm, idx_v, rows_v, sem):
    wid = lax.axis_index("s") * 2 + lax.axis_index("c")
    pltpu.sync_copy(idx_hbm.at[wid], idx_v)
    pltpu.async_copy(x_hbm.at[idx_v], rows_v, sem).wait()
    pltpu.sync_copy(rows_v, out_hbm.at[wid])


def _gather_rows(x2d, gidx):
    mesh = plsc.VectorSubcoreMesh(core_axis_name="c", subcore_axis_name="s")
    f = pl.kernel(
        _sc_gather,
        mesh=mesh,
        out_type=jax.ShapeDtypeStruct((_NW, _RPW, _T), jnp.float32),
        scratch_types=[
            pltpu.VMEM((_RPW,), jnp.int32),
            pltpu.VMEM((_RPW, _T), jnp.float32),
            pltpu.SemaphoreType.DMA,
        ],
    )
    return f(x2d, gidx).reshape(_TOT, _T)


def _bc_step(x_ref, t_ref, xg_ref, sgf_ref, dl_ref, dg_ref, sl_ref, y_ref,
             olog, cis):
    i = pl.program_id(0)
    tm = t_ref[...]

    @pl.when(i == 0)
    def _recurrence():
        olog[...] = jnp.zeros_like(olog)
        sgf = sgf_ref[0, :]

        def body(c, carry):
            dg_c = dg_ref[0, pl.ds(c, 1), :].reshape(_K, 1)
            sl_c = sl_ref[0, pl.ds(c, 1), :].reshape(_K, 1)
            dl_c = dl_ref[0, pl.ds(c, 1), :]

            m = (dg_c == sgf[None, :]).astype(jnp.bfloat16)
            inc = jnp.dot(m, olog[...], preferred_element_type=jnp.float32)
            convinc = jnp.dot(inc, tm, preferred_element_type=jnp.float32)
            cis[pl.ds(c * _K, _K), :] = convinc

            msd = (sl_c == dl_c).astype(jnp.bfloat16)
            xs_conv = jnp.dot(xg_ref[pl.ds(c * _K, _K), :], tm,
                              preferred_element_type=jnp.float32)
            out = xs_conv + jnp.dot(msd, convinc,
                                    preferred_element_type=jnp.float32)
            olog[pl.ds(c * _K, _K), :] = out.astype(jnp.bfloat16)
            return carry

        lax.fori_loop(0, _N_CLUSTERS, body, 0)

    @pl.when(i > 0)
    def _dense():
        b0 = (i - 1) * _CPB
        for j in range(_CPB):
            dl_c = dl_ref[0, pl.ds(b0 + j, 1), :]
            ohd = (lax.broadcasted_iota(jnp.int32, (_CLUSTER, _K), 0)
                   == dl_c).astype(jnp.bfloat16)
            corr = jnp.dot(ohd, cis[pl.ds((b0 + j) * _K, _K), :],
                           preferred_element_type=jnp.float32)
            yj = jnp.dot(x_ref[0, pl.ds(j * _CLUSTER, _CLUSTER), :], tm,
                         preferred_element_type=jnp.float32)
            y_ref[0, pl.ds(j * _CLUSTER, _CLUSTER), :] = yj + corr


def _toeplitz(fir):
    idx = jnp.arange(_T)
    diff = idx[None, :] - idx[:, None]
    mask = (diff >= 0) & (diff < _D)
    return jnp.where(mask, fir[jnp.clip(diff, 0, _D - 1)], 0.0).astype(jnp.float32)


def kernel(x, kernel, dst_local, dst_gidx, src_local, src_gidx):
    fir = kernel
    tmat = _toeplitz(fir)

    sl32 = src_local.astype(jnp.int32)
    gidx = (jnp.arange(_N_CLUSTERS, dtype=jnp.int32)[:, None] * _CLUSTER
            + sl32).reshape(_NW, _RPW)
    xg = _gather_rows(x.reshape(_N_CLUSTERS * _CLUSTER, _T), gidx)

    sgf = src_gidx.astype(jnp.int32).reshape(1, _TOT)
    dl3 = dst_local.astype(jnp.int32).reshape(1, _N_CLUSTERS, _K)
    dg3 = dst_gidx.astype(jnp.int32).reshape(1, _N_CLUSTERS, _K)
    sl3 = sl32.reshape(1, _N_CLUSTERS, _K)

    y = pl.pallas_call(
        _bc_step,
        grid=(_NBLK + 1,),
        in_specs=[
            pl.BlockSpec((1, _CPB * _CLUSTER, _T),
                         lambda i: (0, jnp.maximum(i - 1, 0), 0)),
            pl.BlockSpec((_T, _T), lambda i: (0, 0)),
            pl.BlockSpec((_TOT, _T), lambda i: (0, 0)),
            pl.BlockSpec((1, _TOT), lambda i: (0, 0)),
            pl.BlockSpec((1, _N_CLUSTERS, _K), lambda i: (0, 0, 0)),
            pl.BlockSpec((1, _N_CLUSTERS, _K), lambda i: (0, 0, 0)),
            pl.BlockSpec((1, _N_CLUSTERS, _K), lambda i: (0, 0, 0)),
        ],
        out_specs=pl.BlockSpec((1, _CPB * _CLUSTER, _T),
                               lambda i: (0, jnp.maximum(i - 1, 0), 0)),
        out_shape=jax.ShapeDtypeStruct(x.shape, jnp.float32),
        scratch_shapes=[
            pltpu.VMEM((_TOT, _T), jnp.bfloat16),
            pltpu.VMEM((_TOT, _T), jnp.float32),
        ],
        compiler_params=pltpu.CompilerParams(
            dimension_semantics=("arbitrary",),
        ),
    )(x, tmat, xg, sgf, dl3, dg3, sl3)
    return y
